# baseline (speedup 1.0000x reference)
import functools

import jax
import jax.numpy as jnp
from jax import lax
from jax.experimental import pallas as pl
from jax.experimental.pallas import tpu as pltpu


_EPS = 1e-12


def _stage_a_body(x_ref, ws_ref, wn_ref, ts_ref, tn_ref):
    x = x_ref[...]
    ts_ref[...] = jnp.dot(x, ws_ref[...], preferred_element_type=jnp.float32)
    tn_ref[...] = jnp.dot(x, wn_ref[...], preferred_element_type=jnp.float32)


def _stage_mid_body(ts_ref, aggp_ref, wsump_ref, ws_ref, wn_ref,
                    t1s_ref, t1n_ref):
    agg = aggp_ref[0] + aggp_ref[1]
    wsum = wsump_ref[0] + wsump_ref[1]
    winv = 1.0 / jnp.maximum(wsum, _EPS)
    h = jnp.concatenate([ts_ref[...], agg * winv], axis=-1)
    h = jnp.maximum(h, 0.0)
    norm = jnp.sqrt(jnp.sum(h * h, axis=-1, keepdims=True))
    h = h / jnp.maximum(norm, _EPS)
    t1s_ref[...] = jnp.dot(h, ws_ref[...], preferred_element_type=jnp.float32)
    t1n_ref[...] = jnp.dot(h, wn_ref[...], preferred_element_type=jnp.float32)


def _stage_g_body(ts_ref, aggp_ref, wsump_ref, g_ref):
    agg = aggp_ref[0] + aggp_ref[1]
    wsum = wsump_ref[0] + wsump_ref[1]
    winv = 1.0 / jnp.maximum(wsum, _EPS)
    h = jnp.concatenate([ts_ref[...], agg * winv], axis=-1)
    h = jnp.maximum(h, 0.0)
    norm = jnp.sqrt(jnp.sum(h * h, axis=-1, keepdims=True))
    h = h / jnp.maximum(norm, _EPS)
    dis = lax.rsqrt(wsum + 1.0)
    g_ref[...] = h * dis


def _stage_out_body(g_ref, sp_ref, wsump_ref, wg_ref, bg_ref, out_ref):
    wsum = wsump_ref[0] + wsump_ref[1]
    dis = lax.rsqrt(wsum + 1.0)
    pre = (sp_ref[0] + sp_ref[1] + g_ref[...]) * dis
    out_ref[...] = (jnp.dot(pre, wg_ref[...], preferred_element_type=jnp.float32)
                    + bg_ref[...][None, :])


def _tc_call(body, out_shapes, *args):
    return pl.pallas_call(
        body,
        out_shape=out_shapes,
    )(*args)


_NC = 2
_NS = 16
_NW = _NC * _NS
_K = 128
_NP = 10240
_K2 = 128


def _segsum_body(with_wsum, n, w, e, *refs):
    from jax.experimental.pallas import tpu_sc as plsc

    if with_wsum:
        (table, srcr, dstr, ewr, aggout, wsumout,
         idx_all, didx_all, ew_all, rows0, rows1, rows2, rows3,
         tmp_v, wsum_v, wred_v, wacc_v, acc_sh, wsum16_sh,
         gs0, gs1, gs2, gs3, ss0, ss1, ss2, ss3) = refs
    else:
        (table, srcr, dstr, ewr, aggout,
         idx_all, didx_all, ew_all, rows0, rows1, rows2, rows3,
         tmp_v, acc_sh,
         gs0, gs1, gs2, gs3, ss0, ss1, ss2, ss3) = refs
    rows = (rows0, rows1, rows2, rows3)
    gs = (gs0, gs1, gs2, gs3)
    ss = (ss0, ss1, ss2, ss3)

    c = lax.axis_index("c")
    s = lax.axis_index("s")
    wid = c * _NS + s
    epw = e // _NW
    nch = epw // _K2
    nquad = nch // 4
    rpt = -8 * (-n // (8 * _NS))
    rlast = n - (_NS - 1) * rpt
    zero16 = jnp.zeros((16,), jnp.float32)
    zero16i = jnp.zeros((16,), jnp.int32)

    cbase0 = wid * nch
    pltpu.sync_copy(srcr.at[pl.ds(cbase0, nch)], idx_all.at[pl.ds(0, nch)])
    for r in range(nch, nch + 4):
        for k in range(_K2 // 16):
            idx_all[r, pl.ds(16 * k, 16)] = zero16i
    pltpu.sync_copy(dstr.at[pl.ds(cbase0, nch)], didx_all)
    pltpu.sync_copy(ewr.at[pl.ds(cbase0, nch)], ew_all)

    def zrow(i, carry):
        for k in range(w // 16):
            tmp_v[i, pl.ds(16 * k, 16)] = zero16
        return carry
    lax.fori_loop(0, rpt, zrow, 0)

    @pl.when(s < _NS - 1)
    def _():
        pltpu.sync_copy(tmp_v, acc_sh.at[pl.ds(s * rpt, rpt)])

    @pl.when(s == _NS - 1)
    def _():
        pltpu.sync_copy(tmp_v.at[pl.ds(0, rlast)],
                        acc_sh.at[pl.ds((_NS - 1) * rpt, rlast)])
    if with_wsum:
        def zws(i, carry):
            wsum_v[pl.ds(16 * i, 16)] = zero16
            return carry
        lax.fori_loop(0, _NP // 16, zws, 0)
    plsc.subcore_barrier()

    def issue_gather(j, q):
        pltpu.async_copy(table.at[idx_all.at[j]], rows[q], gs[q])

    def wait_gather(j, q):
        pltpu.make_async_copy(table.at[idx_all.at[j]], rows[q], gs[q]).wait()

    def issue_scatter(j, q):
        pltpu.async_copy(rows[q], acc_sh.at[didx_all.at[j]], ss[q], add=True)

    def wait_scatter(j, q):
        pltpu.make_async_copy(rows[q], acc_sh.at[didx_all.at[j]],
                              ss[q]).wait()

    def mul(j, q):
        for g in range(_K2 // 16):
            ew16 = ew_all[j, pl.ds(16 * g, 16)]
            if with_wsum:
                didx16 = didx_all[j, pl.ds(16 * g, 16)]
                plsc.addupdate_scatter(wsum_v, [didx16], ew16)
            for i in range(16):
                wgt = ew16[i]
                for k in range(w // 16):
                    rows[q][16 * g + i, pl.ds(16 * k, 16)] = (
                        rows[q][16 * g + i, pl.ds(16 * k, 16)] * wgt)

    for q in range(4):
        issue_gather(q, q)

    def quad(p, carry):
        j0 = 4 * p
        for q in range(4):
            wait_gather(j0 + q, q)
            mul(j0 + q, q)
            issue_scatter(j0 + q, q)
        for q in range(4):
            wait_scatter(j0 + q, q)
            issue_gather(j0 + 4 + q, q)
        return carry
    lax.fori_loop(0, nquad, quad, 0)

    for q in range(4):
        wait_gather(nch + q, q)

    if with_wsum:
        pltpu.sync_copy(wsum_v, wsum16_sh.at[s])
    plsc.subcore_barrier()

    @pl.when(s < _NS - 1)
    def _():
        pltpu.sync_copy(acc_sh.at[pl.ds(s * rpt, rpt)], tmp_v)
        pltpu.sync_copy(tmp_v, aggout.at[c, pl.ds(s * rpt, rpt)])

    @pl.when(s == _NS - 1)
    def _():
        pltpu.sync_copy(acc_sh.at[pl.ds((_NS - 1) * rpt, rlast)],
                        tmp_v.at[pl.ds(0, rlast)])
        pltpu.sync_copy(tmp_v.at[pl.ds(0, rlast)],
                        aggout.at[c, pl.ds((_NS - 1) * rpt, rlast)])
    if with_wsum:
        colb = s * (_NP // _NS)
        def zwa(i, carry):
            wacc_v[pl.ds(16 * i, 16)] = zero16
            return carry
        lax.fori_loop(0, (_NP // _NS) // 16, zwa, 0)
        for t in range(_NS):
            pltpu.sync_copy(wsum16_sh.at[t, pl.ds(colb, _NP // _NS)], wred_v)

            def wadd(i, carry):
                wacc_v[pl.ds(16 * i, 16)] = (wacc_v[pl.ds(16 * i, 16)]
                                             + wred_v[pl.ds(16 * i, 16)])
                return carry
            lax.fori_loop(0, (_NP // _NS) // 16, wadd, 0)
        pltpu.sync_copy(wacc_v, wsumout.at[c, pl.ds(colb, _NP // _NS)])


def _segsum(table, src, dst, ew, with_wsum):
    from jax.experimental.pallas import tpu_sc as plsc

    n, w = table.shape
    e0 = src.shape[0]
    grain = _NW * _K2 * 4
    e = -grain * (-e0 // grain)
    if e != e0:
        pad = e - e0
        src = jnp.pad(src, (0, pad))
        dst = jnp.pad(dst, (0, pad))
        ew = jnp.pad(ew, (0, pad))
    epw = e // _NW
    nch = epw // _K2
    out_type = [jax.ShapeDtypeStruct((2, n, w), jnp.float32)]
    scratch = [
        pltpu.VMEM((nch + 4, _K2), jnp.int32),
        pltpu.VMEM((nch, _K2), jnp.int32),
        pltpu.VMEM((nch, _K2), jnp.float32),
        pltpu.VMEM((_K2, w), jnp.float32),
        pltpu.VMEM((_K2, w), jnp.float32),
        pltpu.VMEM((_K2, w), jnp.float32),
        pltpu.VMEM((_K2, w), jnp.float32),
        pltpu.VMEM((-8 * (-n // (8 * _NS)), w), jnp.float32),
    ]
    if with_wsum:
        out_type.append(jax.ShapeDtypeStruct((2, _NP), jnp.float32))
        scratch.append(pltpu.VMEM((_NP,), jnp.float32))
        scratch.append(pltpu.VMEM((_NP // _NS,), jnp.float32))
        scratch.append(pltpu.VMEM((_NP // _NS,), jnp.float32))
    scratch.append(pltpu.VMEM_SHARED((n, w), jnp.float32))
    if with_wsum:
        scratch.append(pltpu.VMEM_SHARED((_NS, _NP), jnp.float32))
    scratch += [pltpu.SemaphoreType.DMA] * 8

    mesh = plsc.VectorSubcoreMesh(core_axis_name="c", subcore_axis_name="s")
    f = pl.kernel(
        functools.partial(_segsum_body, with_wsum, n, w, e),
        out_type=tuple(out_type),
        mesh=mesh,
        scratch_types=scratch,
        compiler_params=pltpu.CompilerParams(use_tc_tiling_on_sc=False,
                                             needs_layout_passes=False),
    )
    src2 = src.reshape(e // _K2, _K2)
    dst2 = dst.reshape(e // _K2, _K2)
    ew2 = ew.reshape(e // _K2, _K2)
    if with_wsum:
        aggp, wsump = f(table, src2, dst2, ew2)
        return aggp, wsump[:, :n].reshape(2, n, 1)
    res = f(table, src2, dst2, ew2)
    return res[0], None


def kernel(x, edge_index, edge_weight, W0_self, W0_neigh, W1_self, W1_neigh,
           Wg, bg):
    src = edge_index[0]
    dst = edge_index[1]
    ew = edge_weight
    n = x.shape[0]

    t0s, t0n = _tc_call(
        _stage_a_body,
        (jax.ShapeDtypeStruct((n, W0_self.shape[1]), jnp.float32),
         jax.ShapeDtypeStruct((n, W0_neigh.shape[1]), jnp.float32)),
        x, W0_self, W0_neigh)

    agg0p, wsump = _segsum(t0n, src, dst, ew, with_wsum=True)

    t1s, t1n = _tc_call(
        _stage_mid_body,
        (jax.ShapeDtypeStruct((n, W1_self.shape[1]), jnp.float32),
         jax.ShapeDtypeStruct((n, W1_neigh.shape[1]), jnp.float32)),
        t0s, agg0p, wsump, W1_self, W1_neigh)

    agg1p, _ = _segsum(t1n, src, dst, ew, with_wsum=False)

    g = _tc_call(
        _stage_g_body,
        jax.ShapeDtypeStruct((n, 2 * W1_self.shape[1]), jnp.float32),
        t1s, agg1p, wsump)

    sp, _ = _segsum(g, src, dst, ew, with_wsum=False)

    out = _tc_call(
        _stage_out_body,
        jax.ShapeDtypeStruct((n, Wg.shape[1]), jnp.float32),
        g, sp, wsump, Wg, bg)
    return out


# speedup vs baseline: 1.0167x; 1.0167x over previous
import functools

import jax
import jax.numpy as jnp
from jax import lax
from jax.experimental import pallas as pl
from jax.experimental.pallas import tpu as pltpu


_EPS = 1e-12


def _stage_a_body(x_ref, ws_ref, wn_ref, ts_ref, tn_ref):
    x = x_ref[...]
    ts_ref[...] = jnp.dot(x, ws_ref[...], preferred_element_type=jnp.float32)
    tn_ref[...] = jnp.dot(x, wn_ref[...], preferred_element_type=jnp.float32)


def _stage_mid_body(ts_ref, aggp_ref, wsump_ref, ws_ref, wn_ref,
                    t1s_ref, t1n_ref):
    agg = aggp_ref[0] + aggp_ref[1]
    wsum = wsump_ref[0] + wsump_ref[1]
    winv = 1.0 / jnp.maximum(wsum, _EPS)
    h = jnp.concatenate([ts_ref[...], agg * winv], axis=-1)
    h = jnp.maximum(h, 0.0)
    norm = jnp.sqrt(jnp.sum(h * h, axis=-1, keepdims=True))
    h = h / jnp.maximum(norm, _EPS)
    t1s_ref[...] = jnp.dot(h, ws_ref[...], preferred_element_type=jnp.float32)
    t1n_ref[...] = jnp.dot(h, wn_ref[...], preferred_element_type=jnp.float32)


def _stage_g_body(ts_ref, aggp_ref, wsump_ref, g_ref):
    agg = aggp_ref[0] + aggp_ref[1]
    wsum = wsump_ref[0] + wsump_ref[1]
    winv = 1.0 / jnp.maximum(wsum, _EPS)
    h = jnp.concatenate([ts_ref[...], agg * winv], axis=-1)
    h = jnp.maximum(h, 0.0)
    norm = jnp.sqrt(jnp.sum(h * h, axis=-1, keepdims=True))
    h = h / jnp.maximum(norm, _EPS)
    dis = lax.rsqrt(wsum + 1.0)
    g_ref[...] = h * dis


def _stage_out_body(g_ref, sp_ref, wsump_ref, wg_ref, bg_ref, out_ref):
    wsum = wsump_ref[0] + wsump_ref[1]
    dis = lax.rsqrt(wsum + 1.0)
    pre = (sp_ref[0] + sp_ref[1] + g_ref[...]) * dis
    out_ref[...] = (jnp.dot(pre, wg_ref[...], preferred_element_type=jnp.float32)
                    + bg_ref[...][None, :])


def _tc_call(body, out_shapes, *args):
    return pl.pallas_call(
        body,
        out_shape=out_shapes,
    )(*args)


_NC = 2
_NS = 16
_NW = _NC * _NS
_K = 128
_NP = 10240
_K2 = 128


def _segsum_body(with_wsum, n, w, e, *refs):
    from jax.experimental.pallas import tpu_sc as plsc

    if with_wsum:
        (table, srcr, dstr, ewr, aggout, wsumout,
         idx_all, didx_all, ew_all, rows0, rows1, rows2, rows3,
         tmp_v, wsum_v, wred_v, wacc_v, acc_sh, wsum16_sh,
         gs0, gs1, gs2, gs3, ss0, ss1, ss2, ss3) = refs
    else:
        (table, srcr, dstr, ewr, aggout,
         idx_all, didx_all, ew_all, rows0, rows1, rows2, rows3,
         tmp_v, acc_sh,
         gs0, gs1, gs2, gs3, ss0, ss1, ss2, ss3) = refs
    rows = (rows0, rows1, rows2, rows3)
    gs = (gs0, gs1, gs2, gs3)
    ss = (ss0, ss1, ss2, ss3)

    c = lax.axis_index("c")
    s = lax.axis_index("s")
    wid = c * _NS + s
    epw = e // _NW
    nch = epw // _K2
    nquad = nch // 4
    rpt = -8 * (-n // (8 * _NS))
    rlast = n - (_NS - 1) * rpt
    zero16 = jnp.zeros((16,), jnp.float32)
    zero16i = jnp.zeros((16,), jnp.int32)

    cbase0 = wid * nch
    pltpu.sync_copy(srcr.at[pl.ds(cbase0, nch)], idx_all.at[pl.ds(0, nch)])
    for r in range(nch, nch + 4):
        for k in range(_K2 // 16):
            idx_all[r, pl.ds(16 * k, 16)] = zero16i
    pltpu.sync_copy(dstr.at[pl.ds(cbase0, nch)], didx_all)
    pltpu.sync_copy(ewr.at[pl.ds(cbase0, nch)], ew_all)

    def zrow(i, carry):
        for k in range(w // 16):
            tmp_v[i, pl.ds(16 * k, 16)] = zero16
        return carry
    lax.fori_loop(0, rpt, zrow, 0)

    @pl.when(s < _NS - 1)
    def _():
        pltpu.sync_copy(tmp_v, acc_sh.at[pl.ds(s * rpt, rpt)])

    @pl.when(s == _NS - 1)
    def _():
        pltpu.sync_copy(tmp_v.at[pl.ds(0, rlast)],
                        acc_sh.at[pl.ds((_NS - 1) * rpt, rlast)])
    if with_wsum:
        def zws(i, carry):
            wsum_v[pl.ds(16 * i, 16)] = zero16
            return carry
        lax.fori_loop(0, _NP // 16, zws, 0)
    plsc.subcore_barrier()

    def issue_gather(j, q):
        pltpu.async_copy(table.at[idx_all.at[j]], rows[q], gs[q])

    def wait_gather(j, q):
        pltpu.make_async_copy(table.at[idx_all.at[j]], rows[q], gs[q]).wait()

    def issue_scatter(j, q):
        pltpu.async_copy(rows[q], acc_sh.at[didx_all.at[j]], ss[q], add=True)

    def wait_scatter(j, q):
        pltpu.make_async_copy(rows[q], acc_sh.at[didx_all.at[j]],
                              ss[q]).wait()

    def mul(j, q):
        for g in range(_K2 // 16):
            ew16 = ew_all[j, pl.ds(16 * g, 16)]
            if with_wsum:
                didx16 = didx_all[j, pl.ds(16 * g, 16)]
                plsc.addupdate_scatter(wsum_v, [didx16], ew16)
            for i in range(16):
                wgt = ew16[i]
                for k in range(w // 16):
                    rows[q][16 * g + i, pl.ds(16 * k, 16)] = (
                        rows[q][16 * g + i, pl.ds(16 * k, 16)] * wgt)

    for q in range(4):
        issue_gather(q, q)

    def quad(p, carry):
        j0 = 4 * p
        for q in range(4):
            wait_gather(j0 + q, q)
            mul(j0 + q, q)
            issue_scatter(j0 + q, q)
        for q in range(4):
            wait_scatter(j0 + q, q)
            issue_gather(j0 + 4 + q, q)
        return carry
    lax.fori_loop(0, nquad, quad, 0)

    for q in range(4):
        wait_gather(nch + q, q)

    if with_wsum:
        pltpu.sync_copy(wsum_v, wsum16_sh.at[s])
    plsc.subcore_barrier()

    @pl.when(s < _NS - 1)
    def _():
        pltpu.sync_copy(acc_sh.at[pl.ds(s * rpt, rpt)], tmp_v)
        pltpu.sync_copy(tmp_v, aggout.at[c, pl.ds(s * rpt, rpt)])

    @pl.when(s == _NS - 1)
    def _():
        pltpu.sync_copy(acc_sh.at[pl.ds((_NS - 1) * rpt, rlast)],
                        tmp_v.at[pl.ds(0, rlast)])
        pltpu.sync_copy(tmp_v.at[pl.ds(0, rlast)],
                        aggout.at[c, pl.ds((_NS - 1) * rpt, rlast)])
    if with_wsum:
        colb = s * (_NP // _NS)
        def zwa(i, carry):
            wacc_v[pl.ds(16 * i, 16)] = zero16
            return carry
        lax.fori_loop(0, (_NP // _NS) // 16, zwa, 0)
        for t in range(_NS):
            pltpu.sync_copy(wsum16_sh.at[t, pl.ds(colb, _NP // _NS)], wred_v)

            def wadd(i, carry):
                wacc_v[pl.ds(16 * i, 16)] = (wacc_v[pl.ds(16 * i, 16)]
                                             + wred_v[pl.ds(16 * i, 16)])
                return carry
            lax.fori_loop(0, (_NP // _NS) // 16, wadd, 0)
        pltpu.sync_copy(wacc_v, wsumout.at[c, pl.ds(colb, _NP // _NS)])


def _segsum(table, src, dst, ew, with_wsum):
    from jax.experimental.pallas import tpu_sc as plsc

    n, w = table.shape
    e0 = src.shape[0]
    grain = _NW * _K2 * 4
    e = -grain * (-e0 // grain)
    if e != e0:
        pad = e - e0
        src = jnp.pad(src, (0, pad))
        dst = jnp.concatenate(
            [dst, (jnp.arange(pad, dtype=dst.dtype) * 37) % n])
        ew = jnp.pad(ew, (0, pad))
    epw = e // _NW
    nch = epw // _K2
    out_type = [jax.ShapeDtypeStruct((2, n, w), jnp.float32)]
    scratch = [
        pltpu.VMEM((nch + 4, _K2), jnp.int32),
        pltpu.VMEM((nch, _K2), jnp.int32),
        pltpu.VMEM((nch, _K2), jnp.float32),
        pltpu.VMEM((_K2, w), jnp.float32),
        pltpu.VMEM((_K2, w), jnp.float32),
        pltpu.VMEM((_K2, w), jnp.float32),
        pltpu.VMEM((_K2, w), jnp.float32),
        pltpu.VMEM((-8 * (-n // (8 * _NS)), w), jnp.float32),
    ]
    if with_wsum:
        out_type.append(jax.ShapeDtypeStruct((2, _NP), jnp.float32))
        scratch.append(pltpu.VMEM((_NP,), jnp.float32))
        scratch.append(pltpu.VMEM((_NP // _NS,), jnp.float32))
        scratch.append(pltpu.VMEM((_NP // _NS,), jnp.float32))
    scratch.append(pltpu.VMEM_SHARED((n, w), jnp.float32))
    if with_wsum:
        scratch.append(pltpu.VMEM_SHARED((_NS, _NP), jnp.float32))
    scratch += [pltpu.SemaphoreType.DMA] * 8

    mesh = plsc.VectorSubcoreMesh(core_axis_name="c", subcore_axis_name="s")
    f = pl.kernel(
        functools.partial(_segsum_body, with_wsum, n, w, e),
        out_type=tuple(out_type),
        mesh=mesh,
        scratch_types=scratch,
        compiler_params=pltpu.CompilerParams(use_tc_tiling_on_sc=False,
                                             needs_layout_passes=False),
    )
    src2 = src.reshape(e // _K2, _K2)
    dst2 = dst.reshape(e // _K2, _K2)
    ew2 = ew.reshape(e // _K2, _K2)
    if with_wsum:
        aggp, wsump = f(table, src2, dst2, ew2)
        return aggp, wsump[:, :n].reshape(2, n, 1)
    res = f(table, src2, dst2, ew2)
    return res[0], None


def kernel(x, edge_index, edge_weight, W0_self, W0_neigh, W1_self, W1_neigh,
           Wg, bg):
    src = edge_index[0]
    dst = edge_index[1]
    ew = edge_weight
    n = x.shape[0]

    t0s, t0n = _tc_call(
        _stage_a_body,
        (jax.ShapeDtypeStruct((n, W0_self.shape[1]), jnp.float32),
         jax.ShapeDtypeStruct((n, W0_neigh.shape[1]), jnp.float32)),
        x, W0_self, W0_neigh)

    agg0p, wsump = _segsum(t0n, src, dst, ew, with_wsum=True)

    t1s, t1n = _tc_call(
        _stage_mid_body,
        (jax.ShapeDtypeStruct((n, W1_self.shape[1]), jnp.float32),
         jax.ShapeDtypeStruct((n, W1_neigh.shape[1]), jnp.float32)),
        t0s, agg0p, wsump, W1_self, W1_neigh)

    agg1p, _ = _segsum(t1n, src, dst, ew, with_wsum=False)

    g = _tc_call(
        _stage_g_body,
        jax.ShapeDtypeStruct((n, 2 * W1_self.shape[1]), jnp.float32),
        t1s, agg1p, wsump)

    sp, _ = _segsum(g, src, dst, ew, with_wsum=False)

    out = _tc_call(
        _stage_out_body,
        jax.ShapeDtypeStruct((n, Wg.shape[1]), jnp.float32),
        g, sp, wsump, Wg, bg)
    return out


# speedup vs baseline: 1.7916x; 1.7622x over previous
import functools

import jax
import jax.numpy as jnp
from jax import lax
from jax.experimental import pallas as pl
from jax.experimental.pallas import tpu as pltpu


_EPS = 1e-12


def _stage_a_body(x_ref, ws_ref, wn_ref, ts_ref, tn_ref):
    x = x_ref[...]
    ts_ref[...] = jnp.dot(x, ws_ref[...], preferred_element_type=jnp.float32)
    tn_ref[...] = jnp.dot(x, wn_ref[...], preferred_element_type=jnp.float32)


def _stage_mid_body(ts_ref, aggp_ref, wsump_ref, ws_ref, wn_ref,
                    t1s_ref, t1n_ref):
    agg = aggp_ref[0] + aggp_ref[1]
    wsum = wsump_ref[0] + wsump_ref[1]
    winv = 1.0 / jnp.maximum(wsum, _EPS)
    h = jnp.concatenate([ts_ref[...], agg * winv], axis=-1)
    h = jnp.maximum(h, 0.0)
    norm = jnp.sqrt(jnp.sum(h * h, axis=-1, keepdims=True))
    h = h / jnp.maximum(norm, _EPS)
    t1s_ref[...] = jnp.dot(h, ws_ref[...], preferred_element_type=jnp.float32)
    t1n_ref[...] = jnp.dot(h, wn_ref[...], preferred_element_type=jnp.float32)


def _stage_g_body(ts_ref, aggp_ref, wsump_ref, g_ref):
    agg = aggp_ref[0] + aggp_ref[1]
    wsum = wsump_ref[0] + wsump_ref[1]
    winv = 1.0 / jnp.maximum(wsum, _EPS)
    h = jnp.concatenate([ts_ref[...], agg * winv], axis=-1)
    h = jnp.maximum(h, 0.0)
    norm = jnp.sqrt(jnp.sum(h * h, axis=-1, keepdims=True))
    h = h / jnp.maximum(norm, _EPS)
    dis = lax.rsqrt(wsum + 1.0)
    g_ref[...] = h * dis


def _stage_out_body(g_ref, sp_ref, wsump_ref, wg_ref, bg_ref, out_ref):
    wsum = wsump_ref[0] + wsump_ref[1]
    dis = lax.rsqrt(wsum + 1.0)
    pre = (sp_ref[0] + sp_ref[1] + g_ref[...]) * dis
    out_ref[...] = (jnp.dot(pre, wg_ref[...], preferred_element_type=jnp.float32)
                    + bg_ref[...][None, :])


def _tc_call(body, out_shapes, *args):
    return pl.pallas_call(
        body,
        out_shape=out_shapes,
    )(*args)


_NC = 2
_NS = 16
_NW = _NC * _NS
_K = 128
_NP = 10240
_K2 = 80


def _segsum_body(with_wsum, n, w, e, *refs):
    from jax.experimental.pallas import tpu_sc as plsc

    if with_wsum:
        (table, srcr, dstr, ewr, aggout, wsumout,
         idx_all, didx_all, ew_all, rows0, rows1, rows2, rows3,
         tmp_v, wsum_v, wred_v, wacc_v, acc_sh, wsum16_sh,
         gs0, gs1, gs2, gs3, ss0, ss1, ss2, ss3) = refs
    else:
        (table, srcr, dstr, ewr, aggout,
         idx_all, didx_all, ew_all, rows0, rows1, rows2, rows3,
         tmp_v, acc_sh,
         gs0, gs1, gs2, gs3, ss0, ss1, ss2, ss3) = refs
    rows = (rows0, rows1, rows2, rows3)
    gs = (gs0, gs1, gs2, gs3)
    ss = (ss0, ss1, ss2, ss3)

    c = lax.axis_index("c")
    s = lax.axis_index("s")
    wid = c * _NS + s
    epw = e // _NW
    nch = epw // _K2
    nquad = (nch - 1) // 4
    rpt = -8 * (-n // (8 * _NS))
    rlast = n - (_NS - 1) * rpt
    zero16 = jnp.zeros((16,), jnp.float32)
    zero16i = jnp.zeros((16,), jnp.int32)

    cbase0 = wid * nch
    pltpu.sync_copy(srcr.at[pl.ds(cbase0, nch)], idx_all.at[pl.ds(0, nch)])
    for r in range(nch, nch + 3):
        for k in range(_K2 // 16):
            idx_all[r, pl.ds(16 * k, 16)] = zero16i
    pltpu.sync_copy(dstr.at[pl.ds(cbase0, nch)], didx_all)
    pltpu.sync_copy(ewr.at[pl.ds(cbase0, nch)], ew_all)

    def zrow(i, carry):
        for k in range(w // 16):
            tmp_v[i, pl.ds(16 * k, 16)] = zero16
        return carry
    lax.fori_loop(0, rpt, zrow, 0)

    @pl.when(s < _NS - 1)
    def _():
        pltpu.sync_copy(tmp_v, acc_sh.at[pl.ds(s * rpt, rpt)])

    @pl.when(s == _NS - 1)
    def _():
        pltpu.sync_copy(tmp_v.at[pl.ds(0, rlast)],
                        acc_sh.at[pl.ds((_NS - 1) * rpt, rlast)])
    if with_wsum:
        def zws(i, carry):
            wsum_v[pl.ds(16 * i, 16)] = zero16
            return carry
        lax.fori_loop(0, _NP // 16, zws, 0)
    plsc.subcore_barrier()

    def issue_gather(j, q):
        pltpu.async_copy(table.at[idx_all.at[j]], rows[q], gs[q])

    def wait_gather(j, q):
        pltpu.make_async_copy(table.at[idx_all.at[j]], rows[q], gs[q]).wait()

    def issue_scatter(j, q):
        pltpu.async_copy(rows[q], acc_sh.at[didx_all.at[j]], ss[q], add=True)

    def wait_scatter(j, q):
        pltpu.make_async_copy(rows[q], acc_sh.at[didx_all.at[j]],
                              ss[q]).wait()

    def mul(j, q):
        for g in range(_K2 // 16):
            ew16 = ew_all[j, pl.ds(16 * g, 16)]
            if with_wsum:
                didx16 = didx_all[j, pl.ds(16 * g, 16)]
                plsc.addupdate_scatter(wsum_v, [didx16], ew16)
            for i in range(16):
                wgt = ew16[i]
                for k in range(w // 16):
                    rows[q][16 * g + i, pl.ds(16 * k, 16)] = (
                        rows[q][16 * g + i, pl.ds(16 * k, 16)] * wgt)

    for q in range(4):
        issue_gather(q, q)

    def quad(p, carry):
        j0 = 4 * p
        for q in range(4):
            wait_gather(j0 + q, q)
            mul(j0 + q, q)
            issue_scatter(j0 + q, q)
        for q in range(4):
            wait_scatter(j0 + q, q)
            issue_gather(j0 + 4 + q, q)
        return carry
    lax.fori_loop(0, nquad, quad, 0)

    jt = nch - 1
    wait_gather(jt, 0)
    mul(jt, 0)
    issue_scatter(jt, 0)
    for q in range(1, 4):
        wait_gather(jt + q, q)
    wait_scatter(jt, 0)

    if with_wsum:
        pltpu.sync_copy(wsum_v, wsum16_sh.at[s])
    plsc.subcore_barrier()

    @pl.when(s < _NS - 1)
    def _():
        pltpu.sync_copy(acc_sh.at[pl.ds(s * rpt, rpt)], tmp_v)
        pltpu.sync_copy(tmp_v, aggout.at[c, pl.ds(s * rpt, rpt)])

    @pl.when(s == _NS - 1)
    def _():
        pltpu.sync_copy(acc_sh.at[pl.ds((_NS - 1) * rpt, rlast)],
                        tmp_v.at[pl.ds(0, rlast)])
        pltpu.sync_copy(tmp_v.at[pl.ds(0, rlast)],
                        aggout.at[c, pl.ds((_NS - 1) * rpt, rlast)])
    if with_wsum:
        colb = s * (_NP // _NS)
        def zwa(i, carry):
            wacc_v[pl.ds(16 * i, 16)] = zero16
            return carry
        lax.fori_loop(0, (_NP // _NS) // 16, zwa, 0)
        for t in range(_NS):
            pltpu.sync_copy(wsum16_sh.at[t, pl.ds(colb, _NP // _NS)], wred_v)

            def wadd(i, carry):
                wacc_v[pl.ds(16 * i, 16)] = (wacc_v[pl.ds(16 * i, 16)]
                                             + wred_v[pl.ds(16 * i, 16)])
                return carry
            lax.fori_loop(0, (_NP // _NS) // 16, wadd, 0)
        pltpu.sync_copy(wacc_v, wsumout.at[c, pl.ds(colb, _NP // _NS)])


def _segsum(table, src, dst, ew, with_wsum):
    from jax.experimental.pallas import tpu_sc as plsc

    n, w = table.shape
    e = src.shape[0]
    epw = e // _NW
    nch = epw // _K2
    out_type = [jax.ShapeDtypeStruct((2, n, w), jnp.float32)]
    scratch = [
        pltpu.VMEM((nch + 3, _K2), jnp.int32),
        pltpu.VMEM((nch, _K2), jnp.int32),
        pltpu.VMEM((nch, _K2), jnp.float32),
        pltpu.VMEM((_K2, w), jnp.float32),
        pltpu.VMEM((_K2, w), jnp.float32),
        pltpu.VMEM((_K2, w), jnp.float32),
        pltpu.VMEM((_K2, w), jnp.float32),
        pltpu.VMEM((-8 * (-n // (8 * _NS)), w), jnp.float32),
    ]
    if with_wsum:
        out_type.append(jax.ShapeDtypeStruct((2, _NP), jnp.float32))
        scratch.append(pltpu.VMEM((_NP,), jnp.float32))
        scratch.append(pltpu.VMEM((_NP // _NS,), jnp.float32))
        scratch.append(pltpu.VMEM((_NP // _NS,), jnp.float32))
    scratch.append(pltpu.VMEM_SHARED((n, w), jnp.float32))
    if with_wsum:
        scratch.append(pltpu.VMEM_SHARED((_NS, _NP), jnp.float32))
    scratch += [pltpu.SemaphoreType.DMA] * 8

    mesh = plsc.VectorSubcoreMesh(core_axis_name="c", subcore_axis_name="s")
    f = pl.kernel(
        functools.partial(_segsum_body, with_wsum, n, w, e),
        out_type=tuple(out_type),
        mesh=mesh,
        scratch_types=scratch,
        compiler_params=pltpu.CompilerParams(use_tc_tiling_on_sc=False,
                                             needs_layout_passes=False),
    )
    src2 = src.reshape(e // _K2, _K2)
    dst2 = dst.reshape(e // _K2, _K2)
    ew2 = ew.reshape(e // _K2, _K2)
    if with_wsum:
        aggp, wsump = f(table, src2, dst2, ew2)
        return aggp, wsump[:, :n].reshape(2, n, 1)
    res = f(table, src2, dst2, ew2)
    return res[0], None


def kernel(x, edge_index, edge_weight, W0_self, W0_neigh, W1_self, W1_neigh,
           Wg, bg):
    src = edge_index[0]
    dst = edge_index[1]
    ew = edge_weight
    n = x.shape[0]

    t0s, t0n = _tc_call(
        _stage_a_body,
        (jax.ShapeDtypeStruct((n, W0_self.shape[1]), jnp.float32),
         jax.ShapeDtypeStruct((n, W0_neigh.shape[1]), jnp.float32)),
        x, W0_self, W0_neigh)

    agg0p, wsump = _segsum(t0n, src, dst, ew, with_wsum=True)

    t1s, t1n = _tc_call(
        _stage_mid_body,
        (jax.ShapeDtypeStruct((n, W1_self.shape[1]), jnp.float32),
         jax.ShapeDtypeStruct((n, W1_neigh.shape[1]), jnp.float32)),
        t0s, agg0p, wsump, W1_self, W1_neigh)

    agg1p, _ = _segsum(t1n, src, dst, ew, with_wsum=False)

    g = _tc_call(
        _stage_g_body,
        jax.ShapeDtypeStruct((n, 2 * W1_self.shape[1]), jnp.float32),
        t1s, agg1p, wsump)

    sp, _ = _segsum(g, src, dst, ew, with_wsum=False)

    out = _tc_call(
        _stage_out_body,
        jax.ShapeDtypeStruct((n, Wg.shape[1]), jnp.float32),
        g, sp, wsump, Wg, bg)
    return out


# speedup vs baseline: 1.8771x; 1.0477x over previous
import functools

import jax
import jax.numpy as jnp
from jax import lax
from jax.experimental import pallas as pl
from jax.experimental.pallas import tpu as pltpu


_EPS = 1e-12


def _stage_a_body(x_ref, ws_ref, wn_ref, ts_ref, tn_ref):
    x = x_ref[...]
    ts_ref[...] = jnp.dot(x, ws_ref[...], preferred_element_type=jnp.float32)
    tn_ref[...] = jnp.dot(x, wn_ref[...], preferred_element_type=jnp.float32)


def _stage_mid_body(ts_ref, aggp_ref, wsump_ref, ws_ref, wn_ref,
                    t1s_ref, t1n_ref):
    agg = aggp_ref[0] + aggp_ref[1]
    wsum = wsump_ref[0] + wsump_ref[1]
    winv = 1.0 / jnp.maximum(wsum, _EPS)
    h = jnp.concatenate([ts_ref[...], agg * winv], axis=-1)
    h = jnp.maximum(h, 0.0)
    norm = jnp.sqrt(jnp.sum(h * h, axis=-1, keepdims=True))
    h = h / jnp.maximum(norm, _EPS)
    t1s_ref[...] = jnp.dot(h, ws_ref[...], preferred_element_type=jnp.float32)
    t1n_ref[...] = jnp.dot(h, wn_ref[...], preferred_element_type=jnp.float32)


def _stage_g_body(ts_ref, aggp_ref, wsump_ref, g_ref):
    agg = aggp_ref[0] + aggp_ref[1]
    wsum = wsump_ref[0] + wsump_ref[1]
    winv = 1.0 / jnp.maximum(wsum, _EPS)
    h = jnp.concatenate([ts_ref[...], agg * winv], axis=-1)
    h = jnp.maximum(h, 0.0)
    norm = jnp.sqrt(jnp.sum(h * h, axis=-1, keepdims=True))
    h = h / jnp.maximum(norm, _EPS)
    dis = lax.rsqrt(wsum + 1.0)
    g_ref[...] = h * dis


def _stage_out_body(g_ref, sp_ref, wsump_ref, wg_ref, bg_ref, out_ref):
    wsum = wsump_ref[0] + wsump_ref[1]
    dis = lax.rsqrt(wsum + 1.0)
    pre = (sp_ref[0] + sp_ref[1] + g_ref[...]) * dis
    out_ref[...] = (jnp.dot(pre, wg_ref[...], preferred_element_type=jnp.float32)
                    + bg_ref[...][None, :])


def _tc_call(body, out_shapes, *args):
    return pl.pallas_call(
        body,
        out_shape=out_shapes,
    )(*args)


_NC = 2
_NS = 16
_NW = _NC * _NS
_K = 128
_NP = 10240
_K2 = 80
_NB = 8


def _segsum_body(with_wsum, n, w, e, *refs):
    from jax.experimental.pallas import tpu_sc as plsc

    if with_wsum:
        (table, srcr, dstr, ewr, aggout, wsumout,
         idx_all, didx_all, ew_all, *rest) = refs
        rows = rest[0:_NB]
        (tmp_v, wsum_v, wred_v, wacc_v, acc_sh, wsum16_sh) = rest[_NB:_NB + 6]
        gs = rest[_NB + 6:2 * _NB + 6]
        ss = rest[2 * _NB + 6:3 * _NB + 6]
    else:
        (table, srcr, dstr, ewr, aggout,
         idx_all, didx_all, ew_all, *rest) = refs
        rows = rest[0:_NB]
        (tmp_v, acc_sh) = rest[_NB:_NB + 2]
        gs = rest[_NB + 2:2 * _NB + 2]
        ss = rest[2 * _NB + 2:3 * _NB + 2]

    c = lax.axis_index("c")
    s = lax.axis_index("s")
    wid = c * _NS + s
    epw = e // _NW
    nch = epw // _K2
    ngrp = (nch - _NB + 3) // _NB
    rpt = -8 * (-n // (8 * _NS))
    rlast = n - (_NS - 1) * rpt
    zero16 = jnp.zeros((16,), jnp.float32)
    zero16i = jnp.zeros((16,), jnp.int32)

    cbase0 = wid * nch
    pltpu.sync_copy(srcr.at[pl.ds(cbase0, nch)], idx_all.at[pl.ds(0, nch)])
    for r in range(nch, nch + _NB - 1):
        for k in range(_K2 // 16):
            idx_all[r, pl.ds(16 * k, 16)] = zero16i
    pltpu.sync_copy(dstr.at[pl.ds(cbase0, nch)], didx_all)
    pltpu.sync_copy(ewr.at[pl.ds(cbase0, nch)], ew_all)

    def zrow(i, carry):
        for k in range(w // 16):
            tmp_v[i, pl.ds(16 * k, 16)] = zero16
        return carry
    lax.fori_loop(0, rpt, zrow, 0)

    @pl.when(s < _NS - 1)
    def _():
        pltpu.sync_copy(tmp_v, acc_sh.at[pl.ds(s * rpt, rpt)])

    @pl.when(s == _NS - 1)
    def _():
        pltpu.sync_copy(tmp_v.at[pl.ds(0, rlast)],
                        acc_sh.at[pl.ds((_NS - 1) * rpt, rlast)])
    if with_wsum:
        def zws(i, carry):
            wsum_v[pl.ds(16 * i, 16)] = zero16
            return carry
        lax.fori_loop(0, _NP // 16, zws, 0)
    plsc.subcore_barrier()

    def issue_gather(j, q):
        pltpu.async_copy(table.at[idx_all.at[j]], rows[q], gs[q])

    def wait_gather(j, q):
        pltpu.make_async_copy(table.at[idx_all.at[j]], rows[q], gs[q]).wait()

    def issue_scatter(j, q):
        pltpu.async_copy(rows[q], acc_sh.at[didx_all.at[j]], ss[q], add=True)

    def wait_scatter(j, q):
        pltpu.make_async_copy(rows[q], acc_sh.at[didx_all.at[j]],
                              ss[q]).wait()

    def mul(j, q):
        def mulg(g, carry):
            ew16 = ew_all[j, pl.ds(16 * g, 16)]
            if with_wsum:
                didx16 = didx_all[j, pl.ds(16 * g, 16)]
                plsc.addupdate_scatter(wsum_v, [didx16], ew16)
            for i in range(16):
                wgt = ew16[i]
                for k in range(w // 16):
                    rows[q][16 * g + i, pl.ds(16 * k, 16)] = (
                        rows[q][16 * g + i, pl.ds(16 * k, 16)] * wgt)
            return carry
        lax.fori_loop(0, _K2 // 16, mulg, 0)

    for q in range(_NB):
        issue_gather(q, q)

    def group(p, carry):
        j0 = _NB * p
        for q in range(_NB):
            wait_gather(j0 + q, q)
            mul(j0 + q, q)
            issue_scatter(j0 + q, q)
        for q in range(_NB):
            wait_scatter(j0 + q, q)
            issue_gather(j0 + _NB + q, q)
        return carry
    lax.fori_loop(0, ngrp, group, 0)

    jt0 = ngrp * _NB
    ntail = nch - jt0
    for q in range(ntail):
        wait_gather(jt0 + q, q)
        mul(jt0 + q, q)
        issue_scatter(jt0 + q, q)
    for q in range(ntail, _NB):
        wait_gather(jt0 + q, q)
    for q in range(ntail):
        wait_scatter(jt0 + q, q)

    if with_wsum:
        pltpu.sync_copy(wsum_v, wsum16_sh.at[s])
    plsc.subcore_barrier()

    @pl.when(s < _NS - 1)
    def _():
        pltpu.sync_copy(acc_sh.at[pl.ds(s * rpt, rpt)], tmp_v)
        pltpu.sync_copy(tmp_v, aggout.at[c, pl.ds(s * rpt, rpt)])

    @pl.when(s == _NS - 1)
    def _():
        pltpu.sync_copy(acc_sh.at[pl.ds((_NS - 1) * rpt, rlast)],
                        tmp_v.at[pl.ds(0, rlast)])
        pltpu.sync_copy(tmp_v.at[pl.ds(0, rlast)],
                        aggout.at[c, pl.ds((_NS - 1) * rpt, rlast)])
    if with_wsum:
        colb = s * (_NP // _NS)
        def zwa(i, carry):
            wacc_v[pl.ds(16 * i, 16)] = zero16
            return carry
        lax.fori_loop(0, (_NP // _NS) // 16, zwa, 0)
        for t in range(_NS):
            pltpu.sync_copy(wsum16_sh.at[t, pl.ds(colb, _NP // _NS)], wred_v)

            def wadd(i, carry):
                wacc_v[pl.ds(16 * i, 16)] = (wacc_v[pl.ds(16 * i, 16)]
                                             + wred_v[pl.ds(16 * i, 16)])
                return carry
            lax.fori_loop(0, (_NP // _NS) // 16, wadd, 0)
        pltpu.sync_copy(wacc_v, wsumout.at[c, pl.ds(colb, _NP // _NS)])


def _segsum(table, src, dst, ew, with_wsum):
    from jax.experimental.pallas import tpu_sc as plsc

    n, w = table.shape
    e = src.shape[0]
    epw = e // _NW
    nch = epw // _K2
    out_type = [jax.ShapeDtypeStruct((2, n, w), jnp.float32)]
    scratch = [
        pltpu.VMEM((nch + _NB - 1, _K2), jnp.int32),
        pltpu.VMEM((nch, _K2), jnp.int32),
        pltpu.VMEM((nch, _K2), jnp.float32),
    ]
    scratch += [pltpu.VMEM((_K2, w), jnp.float32) for _ in range(_NB)]
    scratch.append(pltpu.VMEM((-8 * (-n // (8 * _NS)), w), jnp.float32))
    if with_wsum:
        out_type.append(jax.ShapeDtypeStruct((2, _NP), jnp.float32))
        scratch.append(pltpu.VMEM((_NP,), jnp.float32))
        scratch.append(pltpu.VMEM((_NP // _NS,), jnp.float32))
        scratch.append(pltpu.VMEM((_NP // _NS,), jnp.float32))
    scratch.append(pltpu.VMEM_SHARED((n, w), jnp.float32))
    if with_wsum:
        scratch.append(pltpu.VMEM_SHARED((_NS, _NP), jnp.float32))
    scratch += [pltpu.SemaphoreType.DMA] * (2 * _NB)

    mesh = plsc.VectorSubcoreMesh(core_axis_name="c", subcore_axis_name="s")
    f = pl.kernel(
        functools.partial(_segsum_body, with_wsum, n, w, e),
        out_type=tuple(out_type),
        mesh=mesh,
        scratch_types=scratch,
        compiler_params=pltpu.CompilerParams(use_tc_tiling_on_sc=False,
                                             needs_layout_passes=False),
    )
    src2 = src.reshape(e // _K2, _K2)
    dst2 = dst.reshape(e // _K2, _K2)
    ew2 = ew.reshape(e // _K2, _K2)
    if with_wsum:
        aggp, wsump = f(table, src2, dst2, ew2)
        return aggp, wsump[:, :n].reshape(2, n, 1)
    res = f(table, src2, dst2, ew2)
    return res[0], None


def kernel(x, edge_index, edge_weight, W0_self, W0_neigh, W1_self, W1_neigh,
           Wg, bg):
    src = edge_index[0]
    dst = edge_index[1]
    ew = edge_weight
    n = x.shape[0]

    t0s, t0n = _tc_call(
        _stage_a_body,
        (jax.ShapeDtypeStruct((n, W0_self.shape[1]), jnp.float32),
         jax.ShapeDtypeStruct((n, W0_neigh.shape[1]), jnp.float32)),
        x, W0_self, W0_neigh)

    agg0p, wsump = _segsum(t0n, src, dst, ew, with_wsum=True)

    t1s, t1n = _tc_call(
        _stage_mid_body,
        (jax.ShapeDtypeStruct((n, W1_self.shape[1]), jnp.float32),
         jax.ShapeDtypeStruct((n, W1_neigh.shape[1]), jnp.float32)),
        t0s, agg0p, wsump, W1_self, W1_neigh)

    agg1p, _ = _segsum(t1n, src, dst, ew, with_wsum=False)

    g = _tc_call(
        _stage_g_body,
        jax.ShapeDtypeStruct((n, 2 * W1_self.shape[1]), jnp.float32),
        t1s, agg1p, wsump)

    sp, _ = _segsum(g, src, dst, ew, with_wsum=False)

    out = _tc_call(
        _stage_out_body,
        jax.ShapeDtypeStruct((n, Wg.shape[1]), jnp.float32),
        g, sp, wsump, Wg, bg)
    return out
